# baseline (speedup 1.0000x reference)
import dataclasses
import functools

import jax
import jax.numpy as jnp
from jax import lax
from jax.experimental import pallas as pl
from jax.experimental.pallas import tpu as pltpu
from jax.experimental.pallas import tpu_sc as plsc

VOCAB = 1000000
D = 64
B = 4096
S = 200
EPS = 1e-5

CHUNK = 128
NBUF = 4
N_WORKERS = 32
N_CHUNKS = (B * S) // CHUNK
CHUNKS_PER_W = N_CHUNKS // N_WORKERS
UNROLL = 4


def _rsqrt_nr(x16):
    i = plsc.bitcast(x16, jnp.int32)
    y = plsc.bitcast(jnp.int32(0x5F3759DF) - (i >> 1), jnp.float32)
    y = y * (1.5 - 0.5 * x16 * y * y)
    y = y * (1.5 - 0.5 * x16 * y * y)
    return y


def _ln_rows(rin, rout, g4, b4, iota):
    @pl.loop(0, CHUNK // 16)
    def _(jg):
        j0 = jg * 16
        jr = j0 + iota
        acc = [None] * 4
        acc2 = [None] * 4
        for f in range(D):
            fv = jnp.full((16,), f, jnp.int32)
            x = plsc.load_gather(rin, [jr, fv])
            k = f % 4
            acc[k] = x if acc[k] is None else acc[k] + x
            acc2[k] = x * x if acc2[k] is None else acc2[k] + x * x
        sv = (acc[0] + acc[1]) + (acc[2] + acc[3])
        qv = (acc2[0] + acc2[1]) + (acc2[2] + acc2[3])
        mean = sv * (1.0 / 64.0)
        var = qv * (1.0 / 64.0) - mean * mean
        rstd = _rsqrt_nr(var + EPS)
        for f in range(D):
            fv = jnp.full((16,), f, jnp.int32)
            x = plsc.load_gather(rin, [jr, fv])
            outv = (x - mean) * rstd * g4[f >> 4][f & 15] + b4[f >> 4][f & 15]
            rout[f >> 3, f & 7, pl.ds(j0, 16)] = outv


def _sc_embed_ln(tok2d, table, gamma, beta):
    mesh = plsc.VectorSubcoreMesh(core_axis_name="c", subcore_axis_name="s")
    cp = pltpu.CompilerParams()
    for fld, val in (("needs_layout_passes", False),
                     ("use_tc_tiling_on_sc", False)):
        if fld in pltpu.CompilerParams.__dataclass_fields__:
            cp = dataclasses.replace(cp, **{fld: val})

    @functools.partial(
        pl.kernel,
        mesh=mesh,
        compiler_params=cp,
        out_type=jax.ShapeDtypeStruct((S, 8, 32, 8, 128), jnp.float32),
        scratch_types=(
            [pltpu.VMEM((CHUNKS_PER_W, CHUNK), jnp.int32)]
            + [pltpu.VMEM((CHUNK, 2 * D), jnp.float32) for _ in range(NBUF)]
            + [pltpu.VMEM((8, 8, CHUNK), jnp.float32) for _ in range(NBUF)]
            + [pltpu.VMEM((D,), jnp.float32) for _ in range(2)]
            + [pltpu.SemaphoreType.DMA for _ in range(2 * NBUF)]
        ),
    )
    def k(tok_hbm, table_hbm, gamma_hbm, beta_hbm, out_hbm, *scratch):
        idx_v = scratch[0]
        rin = list(scratch[1:1 + NBUF])
        rout = list(scratch[1 + NBUF:1 + 2 * NBUF])
        gamma_v, beta_v = scratch[1 + 2 * NBUF:3 + 2 * NBUF]
        gsem = list(scratch[3 + 2 * NBUF:3 + 2 * NBUF + NBUF])
        osem = list(scratch[3 + 2 * NBUF + NBUF:])

        wid = lax.axis_index("c") * 16 + lax.axis_index("s")
        base_chunk = wid * CHUNKS_PER_W

        pltpu.sync_copy(tok_hbm.at[pl.ds(base_chunk, CHUNKS_PER_W)], idx_v)
        pltpu.sync_copy(gamma_hbm, gamma_v)
        pltpu.sync_copy(beta_hbm, beta_v)
        g4 = [gamma_v[pl.ds(16 * i, 16)] for i in range(4)]
        b4 = [beta_v[pl.ds(16 * i, 16)] for i in range(4)]
        iota = lax.iota(jnp.int32, 16)

        for b in range(NBUF):
            pltpu.async_copy(table_hbm.at[idx_v.at[b]], rin[b], gsem[b])

        @pl.loop(0, CHUNKS_PER_W, step=NBUF)
        def _(s0):
            for b in range(NBUF):
                s = s0 + b
                n = base_chunk + s
                s_out = n >> 5
                bt = n & 31

                @pl.when(s >= NBUF)
                def _():
                    for ft in range(8):
                        pltpu.make_async_copy(
                            rout[b].at[ft], out_hbm.at[0, ft, 0], osem[b]
                        ).wait()

                pltpu.make_async_copy(
                    table_hbm.at[idx_v.at[s]], rin[b], gsem[b]
                ).wait()

                _ln_rows(rin[b], rout[b], g4, b4, iota)

                for ft in range(8):
                    pltpu.async_copy(
                        rout[b].at[ft], out_hbm.at[s_out, ft, bt], osem[b]
                    )

                @pl.when(s + NBUF < CHUNKS_PER_W)
                def _():
                    pltpu.async_copy(
                        table_hbm.at[idx_v.at[s + NBUF]], rin[b], gsem[b]
                    )

        for b in range(NBUF):
            for ft in range(8):
                pltpu.make_async_copy(
                    rout[b].at[ft], out_hbm.at[0, ft, 0], osem[b]
                ).wait()

    return k(tok2d, table, gamma, beta)


def kernel(token_ids, table, gamma, beta):
    tok2d = jnp.reshape(jnp.transpose(token_ids), (N_CHUNKS, CHUNK))
    tok2d = tok2d.astype(jnp.int32)
    tblp = jnp.pad(table, ((0, 0), (0, D)))
    out5d = _sc_embed_ln(tok2d, tblp, gamma, beta)
    out = jnp.reshape(jnp.transpose(out5d, (2, 4, 0, 1, 3)), (B, S, D))
    return out


# speedup vs baseline: 1.5763x; 1.5763x over previous
import dataclasses
import functools

import jax
import jax.numpy as jnp
from jax import lax
from jax.experimental import pallas as pl
from jax.experimental.pallas import tpu as pltpu
from jax.experimental.pallas import tpu_sc as plsc

VOCAB = 1000000
D = 64
B = 4096
S = 200
EPS = 1e-5

CHUNK = 128
NBUF = 4
N_WORKERS = 32
TOTAL_ROWS = B * S
N_CHUNKS = TOTAL_ROWS // CHUNK
CHUNKS_PER_W = N_CHUNKS // N_WORKERS
UNROLL = 4


def _rsqrt_nr(x16):
    i = plsc.bitcast(x16, jnp.int32)
    y = plsc.bitcast(jnp.int32(0x5F3759DF) - (i >> 1), jnp.float32)
    y = y * (1.5 - 0.5 * x16 * y * y)
    y = y * (1.5 - 0.5 * x16 * y * y)
    return y


def _ln_rows(rin, rout, gvecs, bvecs):
    @pl.loop(0, CHUNK, step=UNROLL)
    def _(r0):
        for dr in range(UNROLL):
            r = r0 + dr
            vs = [rin[r, pl.ds(16 * q, 16)] for q in range(4)]
            sv = (vs[0] + vs[1]) + (vs[2] + vs[3])
            qv = (vs[0] * vs[0] + vs[1] * vs[1]) + (vs[2] * vs[2] + vs[3] * vs[3])
            tot = jnp.sum(sv)
            qtot = jnp.sum(qv)
            mean = tot * (1.0 / 64.0)
            var = qtot * (1.0 / 64.0) - mean * mean
            xv = jnp.broadcast_to(var + EPS, (16,))
            rstd = _rsqrt_nr(xv)
            for q in range(4):
                outv = (vs[q] - mean) * rstd * gvecs[q] + bvecs[q]
                rout[r, pl.ds(16 * q, 16)] = outv


def _sc_embed_ln(tok2d, table, gamma, beta):
    mesh = plsc.VectorSubcoreMesh(core_axis_name="c", subcore_axis_name="s")
    cp = pltpu.CompilerParams()
    for fld, val in (("needs_layout_passes", False),
                     ("use_tc_tiling_on_sc", False)):
        if fld in pltpu.CompilerParams.__dataclass_fields__:
            cp = dataclasses.replace(cp, **{fld: val})

    @functools.partial(
        pl.kernel,
        mesh=mesh,
        compiler_params=cp,
        out_type=jax.ShapeDtypeStruct((TOTAL_ROWS, 2 * D), jnp.float32),
        scratch_types=(
            [pltpu.VMEM((CHUNKS_PER_W, CHUNK), jnp.int32)]
            + [pltpu.VMEM((CHUNK, 2 * D), jnp.float32) for _ in range(NBUF)]
            + [pltpu.VMEM((CHUNK, D), jnp.float32) for _ in range(NBUF)]
            + [pltpu.VMEM((D,), jnp.float32) for _ in range(2)]
            + [pltpu.SemaphoreType.DMA for _ in range(2 * NBUF)]
        ),
    )
    def k(tok_hbm, table_hbm, gamma_hbm, beta_hbm, out_hbm, *scratch):
        idx_v = scratch[0]
        rin = list(scratch[1:1 + NBUF])
        rout = list(scratch[1 + NBUF:1 + 2 * NBUF])
        gamma_v, beta_v = scratch[1 + 2 * NBUF:3 + 2 * NBUF]
        gsem = list(scratch[3 + 2 * NBUF:3 + 2 * NBUF + NBUF])
        osem = list(scratch[3 + 2 * NBUF + NBUF:])

        wid = lax.axis_index("c") * 16 + lax.axis_index("s")
        base_chunk = wid * CHUNKS_PER_W

        pltpu.sync_copy(tok_hbm.at[pl.ds(base_chunk, CHUNKS_PER_W)], idx_v)
        pltpu.sync_copy(gamma_hbm, gamma_v)
        pltpu.sync_copy(beta_hbm, beta_v)
        gvecs = [gamma_v[pl.ds(16 * q, 16)] for q in range(4)]
        bvecs = [beta_v[pl.ds(16 * q, 16)] for q in range(4)]

        for b in range(NBUF):
            pltpu.async_copy(table_hbm.at[idx_v.at[b]], rin[b], gsem[b])

        @pl.loop(0, CHUNKS_PER_W, step=NBUF)
        def _(s0):
            for b in range(NBUF):
                s = s0 + b
                @pl.when(s >= NBUF)
                def _():
                    pltpu.make_async_copy(
                        rout[b],
                        out_hbm.at[pl.ds(0, CHUNK), pl.ds(0, D)],
                        osem[b],
                    ).wait()

                pltpu.make_async_copy(
                    table_hbm.at[idx_v.at[s]], rin[b], gsem[b]
                ).wait()

                _ln_rows(rin[b], rout[b], gvecs, bvecs)

                pltpu.async_copy(
                    rout[b],
                    out_hbm.at[pl.ds((base_chunk + s) * CHUNK, CHUNK),
                               pl.ds(0, D)],
                    osem[b],
                )

                @pl.when(s + NBUF < CHUNKS_PER_W)
                def _():
                    pltpu.async_copy(
                        table_hbm.at[idx_v.at[s + NBUF]], rin[b], gsem[b]
                    )

        for b in range(NBUF):
            pltpu.make_async_copy(
                rout[b], out_hbm.at[pl.ds(0, CHUNK), pl.ds(0, D)], osem[b]
            ).wait()

    return k(tok2d, table, gamma, beta)


def kernel(token_ids, table, gamma, beta):
    tok2d = jnp.reshape(token_ids, (N_CHUNKS, CHUNK)).astype(jnp.int32)
    tblp = jnp.pad(table, ((0, 0), (0, D)))
    out = _sc_embed_ln(tok2d, tblp, gamma, beta)
    return jnp.reshape(out[:, :D], (B, S, D))


# speedup vs baseline: 2.8370x; 1.7997x over previous
import dataclasses
import functools

import jax
import jax.numpy as jnp
from jax import lax
from jax.experimental import pallas as pl
from jax.experimental.pallas import tpu as pltpu
from jax.experimental.pallas import tpu_sc as plsc

VOCAB = 1000000
D = 64
B = 4096
S = 200
EPS = 1e-5

CHUNK = 128
NBUF = 2
N_WORKERS = 32
TOTAL_ROWS = B * S
N_CHUNKS = TOTAL_ROWS // CHUNK
CHUNKS_PER_W = N_CHUNKS // N_WORKERS
UNROLL = 4


def _rsqrt_nr(x16):
    i = plsc.bitcast(x16, jnp.int32)
    y = plsc.bitcast(jnp.int32(0x5F3759DF) - (i >> 1), jnp.float32)
    y = y * (1.5 - 0.5 * x16 * y * y)
    y = y * (1.5 - 0.5 * x16 * y * y)
    return y


def _ln_rows(rin, rout, gvecs, bvecs):
    @pl.loop(0, CHUNK, step=UNROLL)
    def _(r0):
        for dr in range(UNROLL):
            r = r0 + dr
            vs = [rin[r, pl.ds(16 * q, 16)] for q in range(4)]
            sv = (vs[0] + vs[1]) + (vs[2] + vs[3])
            qv = (vs[0] * vs[0] + vs[1] * vs[1]) + (vs[2] * vs[2] + vs[3] * vs[3])
            tot = jnp.sum(sv)
            qtot = jnp.sum(qv)
            mean = tot * (1.0 / 64.0)
            var = qtot * (1.0 / 64.0) - mean * mean
            xv = jnp.broadcast_to(var + EPS, (16,))
            rstd = _rsqrt_nr(xv)
            for q in range(4):
                outv = (vs[q] - mean) * rstd * gvecs[q] + bvecs[q]
                rout[r, pl.ds(16 * q, 16)] = outv


def _sc_embed_ln(tok2d, table, gamma, beta):
    mesh = plsc.VectorSubcoreMesh(core_axis_name="c", subcore_axis_name="s")
    cp = pltpu.CompilerParams()
    for fld, val in (("needs_layout_passes", False),
                     ("use_tc_tiling_on_sc", False)):
        if fld in pltpu.CompilerParams.__dataclass_fields__:
            cp = dataclasses.replace(cp, **{fld: val})

    @functools.partial(
        pl.kernel,
        mesh=mesh,
        compiler_params=cp,
        out_type=jax.ShapeDtypeStruct((TOTAL_ROWS, 2 * D), jnp.float32),
        scratch_types=(
            [pltpu.VMEM((CHUNKS_PER_W, CHUNK), jnp.int32)]
            + [pltpu.VMEM((CHUNK, 2 * D), jnp.float32) for _ in range(2 * NBUF)]
            + [pltpu.VMEM((D,), jnp.float32) for _ in range(2)]
            + [pltpu.SemaphoreType.DMA for _ in range(2 * NBUF)]
        ),
    )
    def k(tok_hbm, table_hbm, gamma_hbm, beta_hbm, out_hbm, *scratch):
        idx_v = scratch[0]
        rin = list(scratch[1:1 + NBUF])
        rout = list(scratch[1 + NBUF:1 + 2 * NBUF])
        gamma_v, beta_v = scratch[1 + 2 * NBUF:3 + 2 * NBUF]
        gsem = list(scratch[3 + 2 * NBUF:3 + 2 * NBUF + NBUF])
        osem = list(scratch[3 + 2 * NBUF + NBUF:])

        wid = lax.axis_index("c") * 16 + lax.axis_index("s")
        base_chunk = wid * CHUNKS_PER_W

        pltpu.sync_copy(tok_hbm.at[pl.ds(base_chunk, CHUNKS_PER_W)], idx_v)
        pltpu.sync_copy(gamma_hbm, gamma_v)
        pltpu.sync_copy(beta_hbm, beta_v)
        gvecs = [gamma_v[pl.ds(16 * q, 16)] for q in range(4)]
        bvecs = [beta_v[pl.ds(16 * q, 16)] for q in range(4)]

        for b in range(NBUF):
            pltpu.async_copy(table_hbm.at[idx_v.at[b]], rin[b], gsem[b])

        @pl.loop(0, CHUNKS_PER_W, step=NBUF)
        def _(s0):
            for b in range(NBUF):
                s = s0 + b
                @pl.when(s >= NBUF)
                def _():
                    pltpu.make_async_copy(
                        rout[b], out_hbm.at[pl.ds(0, CHUNK)], osem[b]
                    ).wait()

                pltpu.make_async_copy(
                    table_hbm.at[idx_v.at[s]], rin[b], gsem[b]
                ).wait()

                _ln_rows(rin[b], rout[b], gvecs, bvecs)

                pltpu.async_copy(
                    rout[b],
                    out_hbm.at[pl.ds((base_chunk + s) * CHUNK, CHUNK)],
                    osem[b],
                )

                @pl.when(s + NBUF < CHUNKS_PER_W)
                def _():
                    pltpu.async_copy(
                        table_hbm.at[idx_v.at[s + NBUF]], rin[b], gsem[b]
                    )

        for b in range(NBUF):
            pltpu.make_async_copy(
                rout[b], out_hbm.at[pl.ds(0, CHUNK)], osem[b]
            ).wait()

    return k(tok2d, table, gamma, beta)


def kernel(token_ids, table, gamma, beta):
    tok2d = jnp.reshape(token_ids, (N_CHUNKS, CHUNK)).astype(jnp.int32)
    tblp = jnp.pad(table, ((0, 0), (0, D)))
    out = _sc_embed_ln(tok2d, tblp, gamma, beta)
    return jnp.reshape(out[:, :D], (B, S, D))
